# baseline (speedup 1.0000x reference)
import functools

import jax
import jax.numpy as jnp
from jax import lax
from jax.experimental import pallas as pl
from jax.experimental.pallas import tpu as pltpu
from jax.experimental.pallas import tpu_sc as plsc

N, E, H = 10000, 320000, 128
NC, NS = 2, 16
CH = 80
EPW = E // (NC * NS)
CPW = EPW // CH
NPT = 624
NREM = N - NS * NPT
CPS_LIST = (25, 25, 25, 25, 25)
K = len(CPS_LIST)
HP = H // 2

_MESH = plsc.VectorSubcoreMesh(core_axis_name="c", subcore_axis_name="s")


@functools.partial(
    pl.kernel,
    mesh=_MESH,
    out_type=jax.ShapeDtypeStruct((NC, N, H), jnp.float32),
    scratch_types=[
        pltpu.VMEM((CPW, CH), jnp.int32),
        pltpu.VMEM((CH, H), jnp.float32),
        pltpu.VMEM((CH, H), jnp.float32),
        pltpu.VMEM_SHARED((N, H), jnp.float32),
        pltpu.SemaphoreType.DMA,
        pltpu.SemaphoreType.DMA,
    ],
)
def _seg_sum_sc(e_hbm, end_hbm, zeros_hbm, out_hbm, idx_v, rows0_v, rows1_v,
                acc_sh, sem0, sem1):
    c = lax.axis_index("c")
    s = lax.axis_index("s")
    pltpu.sync_copy(zeros_hbm.at[pl.ds(s * NPT, NPT)],
                    acc_sh.at[pl.ds(s * NPT, NPT)])

    @pl.when(s == NS - 1)
    def _zero_tail():
        pltpu.sync_copy(zeros_hbm.at[pl.ds(NS * NPT, NREM)],
                        acc_sh.at[pl.ds(NS * NPT, NREM)])

    pltpu.sync_copy(end_hbm.at[c, s], idx_v)
    plsc.subcore_barrier()
    wbase = (c * NS + s) * EPW

    def _start_in(i, rows, sem):
        pltpu.async_copy(e_hbm.at[pl.ds(wbase + i * CH, CH)], rows, sem)

    def _finish(i, rows, sem):
        pltpu.make_async_copy(e_hbm.at[pl.ds(wbase + i * CH, CH)],
                              rows, sem).wait()
        pltpu.sync_copy(rows, acc_sh.at[idx_v.at[i]], add=True)

    _start_in(0, rows0_v, sem0)

    def body(i, carry):
        b = lax.rem(i, 2)

        @pl.when(i + 1 < CPW)
        def _prefetch():
            @pl.when(b == 0)
            def _():
                _start_in(i + 1, rows1_v, sem1)

            @pl.when(b == 1)
            def _():
                _start_in(i + 1, rows0_v, sem0)

        @pl.when(b == 0)
        def _():
            _finish(i, rows0_v, sem0)

        @pl.when(b == 1)
        def _():
            _finish(i, rows1_v, sem1)

        return carry

    lax.fori_loop(0, CPW, body, 0)
    plsc.subcore_barrier()
    pltpu.sync_copy(acc_sh.at[pl.ds(s * NPT, NPT)],
                    out_hbm.at[c, pl.ds(s * NPT, NPT)])

    @pl.when(s == NS - 1)
    def _write_tail():
        pltpu.sync_copy(acc_sh.at[pl.ds(NS * NPT, NREM)],
                        out_hbm.at[c, pl.ds(NS * NPT, NREM)])


@functools.lru_cache(maxsize=None)
def _make_gather_sc(cps):
    slc = cps * CH * NC * NS
    eps = cps * CH

    @functools.partial(
        pl.kernel,
        mesh=_MESH,
        compiler_params=pltpu.CompilerParams(use_tc_tiling_on_sc=False),
        out_type=jax.ShapeDtypeStruct((slc, HP), jnp.uint32),
        scratch_types=[
            pltpu.VMEM((cps, CH), jnp.int32),
            pltpu.VMEM((cps, CH), jnp.int32),
            pltpu.VMEM((CH, HP), jnp.uint32),
            pltpu.VMEM((CH, HP), jnp.uint32),
            pltpu.VMEM((CH, HP), jnp.uint32),
            pltpu.VMEM((CH, HP), jnp.uint32),
            pltpu.SemaphoreType.DMA,
            pltpu.SemaphoreType.DMA,
            pltpu.SemaphoreType.DMA,
            pltpu.SemaphoreType.DMA,
        ],
    )
    def _gather_sc(ga_hbm, gb_hbm, start_hbm, end_hbm, out_hbm,
                   idxs_v, idxe_v, bufa0_v, bufb0_v, bufa1_v, bufb1_v,
                   sa0, sb0, sa1, sb1):
        c = lax.axis_index("c")
        s = lax.axis_index("s")
        pltpu.sync_copy(start_hbm.at[c, s], idxs_v)
        pltpu.sync_copy(end_hbm.at[c, s], idxe_v)
        wbase = (c * NS + s) * eps

        def _start_in(i, ba, bb, sa, sb):
            pltpu.async_copy(ga_hbm.at[idxs_v.at[i]], ba, sa)
            pltpu.async_copy(gb_hbm.at[idxe_v.at[i]], bb, sb)

        def _finish(i, ba, bb, sa, sb):
            pltpu.make_async_copy(ga_hbm.at[idxs_v.at[i]], ba, sa).wait()
            pltpu.make_async_copy(gb_hbm.at[idxe_v.at[i]], bb, sb).wait()

            MSK_HI = jnp.uint32(0xFFFF0000)
            RND = jnp.uint32(0x7FFF)
            ONE = jnp.uint32(1)

            def add_row(r, carry):
                for k in range(HP // 16):
                    sl = pl.ds(k * 16, 16)
                    pa = ba[r, sl]
                    pb = bb[r, sl]
                    slo = lax.bitcast_convert_type(pa << 16, jnp.float32) \
                        + lax.bitcast_convert_type(pb << 16, jnp.float32)
                    shi = lax.bitcast_convert_type(pa & MSK_HI, jnp.float32) \
                        + lax.bitcast_convert_type(pb & MSK_HI, jnp.float32)
                    ulo = lax.bitcast_convert_type(slo, jnp.uint32)
                    uhi = lax.bitcast_convert_type(shi, jnp.uint32)
                    rlo = (ulo + RND + ((ulo >> 16) & ONE)) >> 16
                    rhi = ((uhi + RND + ((uhi >> 16) & ONE)) >> 16) << 16
                    ba[r, sl] = rlo | rhi
                return carry

            lax.fori_loop(0, CH, add_row, 0)
            pltpu.sync_copy(ba, out_hbm.at[pl.ds(wbase + i * CH, CH)])

        _start_in(0, bufa0_v, bufb0_v, sa0, sb0)

        def body(i, carry):
            b = lax.rem(i, 2)

            @pl.when(i + 1 < cps)
            def _prefetch():
                @pl.when(b == 0)
                def _():
                    _start_in(i + 1, bufa1_v, bufb1_v, sa1, sb1)

                @pl.when(b == 1)
                def _():
                    _start_in(i + 1, bufa0_v, bufb0_v, sa0, sb0)

            @pl.when(b == 0)
            def _():
                _finish(i, bufa0_v, bufb0_v, sa0, sb0)

            @pl.when(b == 1)
            def _():
                _finish(i, bufa1_v, bufb1_v, sa1, sb1)

            return carry

        lax.fori_loop(0, cps, body, 0)

    return _gather_sc


def _pack_bf16(v):
    u = lax.bitcast_convert_type(v, jnp.uint32)
    r = (u + jnp.uint32(0x7FFF) + ((u >> 16) & jnp.uint32(1))) >> 16
    return r[:, :HP] | (r[:, HP:] << 16)


def _node_body(x_ref, m_ref, w1a_ref, w1b_ref, b1_ref, w2_ref, b2_ref,
               wa_ref, wb_ref, be_ref, xout_ref, ga_ref, gb_ref):
    xb = x_ref[...]
    m = m_ref[0] + m_ref[1]
    h = jnp.maximum(
        jnp.dot(xb, w1a_ref[...], preferred_element_type=jnp.float32)
        + jnp.dot(m, w1b_ref[...], preferred_element_type=jnp.float32)
        + b1_ref[...], 0.0)
    xo = jnp.dot(h, w2_ref[...], preferred_element_type=jnp.float32) \
        + b2_ref[...] + xb
    xout_ref[...] = xo
    ga_ref[...] = _pack_bf16(
        jnp.dot(xo, wa_ref[...], preferred_element_type=jnp.float32)
        + be_ref[...])
    gb_ref[...] = _pack_bf16(
        jnp.dot(xo, wb_ref[...], preferred_element_type=jnp.float32))


def _node_tc(x, msgs, w1a, w1b, b1n, w2n, b2n, wea, web, b1e):
    BN = 1000
    grid = (N // BN,)
    row_spec = pl.BlockSpec((BN, H), lambda i: (i, 0))
    w_spec = pl.BlockSpec((H, H), lambda i: (0, 0))
    b_spec = pl.BlockSpec((1, H), lambda i: (0, 0))
    return pl.pallas_call(
        _node_body,
        grid=grid,
        in_specs=[
            row_spec,
            pl.BlockSpec((NC, BN, H), lambda i: (0, i, 0)),
            w_spec, w_spec, b_spec, w_spec, b_spec, w_spec, w_spec, b_spec,
        ],
        out_specs=[row_spec,
                   pl.BlockSpec((BN, HP), lambda i: (i, 0)),
                   pl.BlockSpec((BN, HP), lambda i: (i, 0))],
        out_shape=[jax.ShapeDtypeStruct((N, H), jnp.float32),
                   jax.ShapeDtypeStruct((N, HP), jnp.uint32),
                   jax.ShapeDtypeStruct((N, HP), jnp.uint32)],
    )(x, msgs, w1a, w1b, b1n, w2n, b2n, wea, web, b1e)


def _edge_body(gsum_ref, e_ref, wc_ref, w2_ref, b2_ref, out_ref):
    eb = e_ref[...]
    p = gsum_ref[...]
    lo = lax.bitcast_convert_type(p << 16, jnp.float32)
    hi = lax.bitcast_convert_type(p & jnp.uint32(0xFFFF0000), jnp.float32)
    g = jnp.concatenate([lo, hi], axis=1)
    h = jnp.maximum(
        g + jnp.dot(eb, wc_ref[...], preferred_element_type=jnp.float32), 0.0)
    out_ref[...] = jnp.dot(h, w2_ref[...],
                           preferred_element_type=jnp.float32) \
        + b2_ref[...] + eb


def _edge_body_acc(gsum_ref, e_ref, wc_ref, w2_ref, b2_ref, prev_ref, out_ref):
    del prev_ref
    _edge_body(gsum_ref, e_ref, wc_ref, w2_ref, b2_ref, out_ref)


_BE = 2000


def _edge_tc(row0, gsum_k, e, wec, w2e, b2e, prev):
    blk0 = row0 // _BE
    nblk = gsum_k.shape[0] // _BE
    loc_spec = pl.BlockSpec((_BE, HP), lambda i: (i, 0))
    shift_spec = pl.BlockSpec((_BE, H), lambda i: (i + blk0, 0))
    w_spec = pl.BlockSpec((H, H), lambda i: (0, 0))
    b_spec = pl.BlockSpec((1, H), lambda i: (0, 0))
    body = _edge_body if prev is None else _edge_body_acc
    in_specs = [loc_spec, shift_spec, w_spec, w_spec, b_spec]
    args = [gsum_k, e, wec, w2e, b2e]
    aliases = {}
    if prev is not None:
        in_specs.append(pl.BlockSpec(memory_space=pl.ANY))
        args.append(prev)
        aliases = {5: 0}
    return pl.pallas_call(
        body,
        grid=(nblk,),
        in_specs=in_specs,
        out_specs=shift_spec,
        out_shape=jax.ShapeDtypeStruct((E, H), jnp.float32),
        input_output_aliases=aliases,
    )(*args)


def kernel(x, edge_index, e, W1n, b1n, W2n, b2n, W1e, b1e, W2e, b2e):
    end = edge_index[1].reshape(NC, NS, CPW, CH)
    zeros = jnp.zeros((N, H), jnp.float32)

    msgs = _seg_sum_sc(e, end, zeros)

    x_out, ga_nodes, gb_nodes = _node_tc(
        x, msgs,
        W1n[:H], W1n[H:], b1n.reshape(1, H), W2n, b2n.reshape(1, H),
        W1e[:H], W1e[H:2 * H], b1e.reshape(1, H))

    wec = W1e[2 * H:]
    b2e_r = b2e.reshape(1, H)
    gsums, row0s = [], []
    row0 = 0
    for cps in CPS_LIST:
        slc = cps * CH * NC * NS
        s_idx = lax.slice_in_dim(edge_index[0], row0, row0 + slc) \
            .reshape(NC, NS, cps, CH)
        e_idx = lax.slice_in_dim(edge_index[1], row0, row0 + slc) \
            .reshape(NC, NS, cps, CH)
        gsums.append(_make_gather_sc(cps)(ga_nodes, gb_nodes, s_idx, e_idx))
        row0s.append(row0)
        row0 += slc
    e_out = None
    for k in range(K):
        e_out = _edge_tc(row0s[k], gsums[k], e, wec, W2e, b2e_r, e_out)
    return (x_out, e_out)


# speedup vs baseline: 1.1810x; 1.1810x over previous
import functools

import jax
import jax.numpy as jnp
from jax import lax
from jax.experimental import pallas as pl
from jax.experimental.pallas import tpu as pltpu
from jax.experimental.pallas import tpu_sc as plsc

N, E, H = 10000, 320000, 128
NC, NS = 2, 16
CH = 80
EPW = E // (NC * NS)
CPW = EPW // CH
NPT = 624
NREM = N - NS * NPT
CPS_LIST = (25, 25, 25, 25, 25)
K = len(CPS_LIST)
HP = H // 2

_MESH = plsc.VectorSubcoreMesh(core_axis_name="c", subcore_axis_name="s")


@functools.partial(
    pl.kernel,
    mesh=_MESH,
    out_type=jax.ShapeDtypeStruct((NC, N, H), jnp.float32),
    scratch_types=[
        pltpu.VMEM((CPW, CH), jnp.int32),
        pltpu.VMEM((CH, H), jnp.float32),
        pltpu.VMEM((CH, H), jnp.float32),
        pltpu.VMEM_SHARED((N, H), jnp.float32),
        pltpu.SemaphoreType.DMA,
        pltpu.SemaphoreType.DMA,
    ],
)
def _seg_sum_sc(e_hbm, end_hbm, zeros_hbm, out_hbm, idx_v, rows0_v, rows1_v,
                acc_sh, sem0, sem1):
    c = lax.axis_index("c")
    s = lax.axis_index("s")
    pltpu.sync_copy(zeros_hbm.at[pl.ds(s * NPT, NPT)],
                    acc_sh.at[pl.ds(s * NPT, NPT)])

    @pl.when(s == NS - 1)
    def _zero_tail():
        pltpu.sync_copy(zeros_hbm.at[pl.ds(NS * NPT, NREM)],
                        acc_sh.at[pl.ds(NS * NPT, NREM)])

    pltpu.sync_copy(end_hbm.at[c, s], idx_v)
    plsc.subcore_barrier()
    wbase = (c * NS + s) * EPW

    def _start_in(i, rows, sem):
        pltpu.async_copy(e_hbm.at[pl.ds(wbase + i * CH, CH)], rows, sem)

    def _finish(i, rows, sem):
        pltpu.make_async_copy(e_hbm.at[pl.ds(wbase + i * CH, CH)],
                              rows, sem).wait()
        pltpu.sync_copy(rows, acc_sh.at[idx_v.at[i]], add=True)

    _start_in(0, rows0_v, sem0)

    def body(i, carry):
        b = lax.rem(i, 2)

        @pl.when(i + 1 < CPW)
        def _prefetch():
            @pl.when(b == 0)
            def _():
                _start_in(i + 1, rows1_v, sem1)

            @pl.when(b == 1)
            def _():
                _start_in(i + 1, rows0_v, sem0)

        @pl.when(b == 0)
        def _():
            _finish(i, rows0_v, sem0)

        @pl.when(b == 1)
        def _():
            _finish(i, rows1_v, sem1)

        return carry

    lax.fori_loop(0, CPW, body, 0)
    plsc.subcore_barrier()
    pltpu.sync_copy(acc_sh.at[pl.ds(s * NPT, NPT)],
                    out_hbm.at[c, pl.ds(s * NPT, NPT)])

    @pl.when(s == NS - 1)
    def _write_tail():
        pltpu.sync_copy(acc_sh.at[pl.ds(NS * NPT, NREM)],
                        out_hbm.at[c, pl.ds(NS * NPT, NREM)])


@functools.lru_cache(maxsize=None)
def _make_gather_sc(cps):
    slc = cps * CH * NC * NS
    eps = cps * CH

    @functools.partial(
        pl.kernel,
        mesh=_MESH,
        out_type=jax.ShapeDtypeStruct((slc, HP), jnp.uint32),
        scratch_types=[
            pltpu.VMEM((cps, CH), jnp.int32),
            pltpu.VMEM((cps, CH), jnp.int32),
            pltpu.VMEM((CH, H), jnp.uint32),
            pltpu.VMEM((CH, H), jnp.uint32),
            pltpu.VMEM((CH, H), jnp.uint32),
            pltpu.VMEM((CH, H), jnp.uint32),
            pltpu.VMEM((CH, HP), jnp.uint32),
            pltpu.VMEM((CH, HP), jnp.uint32),
            pltpu.SemaphoreType.DMA,
            pltpu.SemaphoreType.DMA,
            pltpu.SemaphoreType.DMA,
            pltpu.SemaphoreType.DMA,
        ],
    )
    def _gather_sc(gpk_hbm, start_hbm, end_hbm, out_hbm,
                   idxs_v, idxe_v, bufa0_v, bufb0_v, bufa1_v, bufb1_v,
                   bufo0_v, bufo1_v, sa0, sb0, sa1, sb1):
        c = lax.axis_index("c")
        s = lax.axis_index("s")
        pltpu.sync_copy(start_hbm.at[c, s], idxs_v)
        pltpu.sync_copy(end_hbm.at[c, s], idxe_v)
        wbase = (c * NS + s) * eps

        def _start_in(i, ba, bb, sa, sb):
            pltpu.async_copy(gpk_hbm.at[idxs_v.at[i]], ba, sa)
            pltpu.async_copy(gpk_hbm.at[idxe_v.at[i]], bb, sb)

        def _finish(i, ba, bb, bo, sa, sb):
            pltpu.make_async_copy(gpk_hbm.at[idxs_v.at[i]], ba, sa).wait()
            pltpu.make_async_copy(gpk_hbm.at[idxe_v.at[i]], bb, sb).wait()

            MSK_HI = jnp.uint32(0xFFFF0000)
            RND = jnp.uint32(0x7FFF)
            ONE = jnp.uint32(1)

            def add_row(r, carry):
                for k in range(HP // 16):
                    sl = pl.ds(k * 16, 16)
                    pa = ba[r, sl]
                    pb = bb[r, pl.ds(HP + k * 16, 16)]
                    slo = lax.bitcast_convert_type(pa << 16, jnp.float32) \
                        + lax.bitcast_convert_type(pb << 16, jnp.float32)
                    shi = lax.bitcast_convert_type(pa & MSK_HI, jnp.float32) \
                        + lax.bitcast_convert_type(pb & MSK_HI, jnp.float32)
                    ulo = lax.bitcast_convert_type(slo, jnp.uint32)
                    uhi = lax.bitcast_convert_type(shi, jnp.uint32)
                    rlo = (ulo + RND + ((ulo >> 16) & ONE)) >> 16
                    rhi = ((uhi + RND + ((uhi >> 16) & ONE)) >> 16) << 16
                    bo[r, sl] = rlo | rhi
                return carry

            lax.fori_loop(0, CH, add_row, 0)
            pltpu.sync_copy(bo, out_hbm.at[pl.ds(wbase + i * CH, CH)])

        _start_in(0, bufa0_v, bufb0_v, sa0, sb0)

        def body(i, carry):
            b = lax.rem(i, 2)

            @pl.when(i + 1 < cps)
            def _prefetch():
                @pl.when(b == 0)
                def _():
                    _start_in(i + 1, bufa1_v, bufb1_v, sa1, sb1)

                @pl.when(b == 1)
                def _():
                    _start_in(i + 1, bufa0_v, bufb0_v, sa0, sb0)

            @pl.when(b == 0)
            def _():
                _finish(i, bufa0_v, bufb0_v, bufo0_v, sa0, sb0)

            @pl.when(b == 1)
            def _():
                _finish(i, bufa1_v, bufb1_v, bufo1_v, sa1, sb1)

            return carry

        lax.fori_loop(0, cps, body, 0)

    return _gather_sc


def _pack_bf16(v):
    u = lax.bitcast_convert_type(v, jnp.uint32)
    r = (u + jnp.uint32(0x7FFF) + ((u >> 16) & jnp.uint32(1))) >> 16
    return r[:, :HP] | (r[:, HP:] << 16)


def _node_body(x_ref, m_ref, w1a_ref, w1b_ref, b1_ref, w2_ref, b2_ref,
               wa_ref, wb_ref, be_ref, xout_ref, gpk_ref):
    xb = x_ref[...]
    m = m_ref[0] + m_ref[1]
    h = jnp.maximum(
        jnp.dot(xb, w1a_ref[...], preferred_element_type=jnp.float32)
        + jnp.dot(m, w1b_ref[...], preferred_element_type=jnp.float32)
        + b1_ref[...], 0.0)
    xo = jnp.dot(h, w2_ref[...], preferred_element_type=jnp.float32) \
        + b2_ref[...] + xb
    xout_ref[...] = xo
    ga_pk = _pack_bf16(
        jnp.dot(xo, wa_ref[...], preferred_element_type=jnp.float32)
        + be_ref[...])
    gb_pk = _pack_bf16(
        jnp.dot(xo, wb_ref[...], preferred_element_type=jnp.float32))
    gpk_ref[...] = jnp.concatenate([ga_pk, gb_pk], axis=1)


def _node_tc(x, msgs, w1a, w1b, b1n, w2n, b2n, wea, web, b1e):
    BN = 1000
    grid = (N // BN,)
    row_spec = pl.BlockSpec((BN, H), lambda i: (i, 0))
    w_spec = pl.BlockSpec((H, H), lambda i: (0, 0))
    b_spec = pl.BlockSpec((1, H), lambda i: (0, 0))
    return pl.pallas_call(
        _node_body,
        grid=grid,
        in_specs=[
            row_spec,
            pl.BlockSpec((NC, BN, H), lambda i: (0, i, 0)),
            w_spec, w_spec, b_spec, w_spec, b_spec, w_spec, w_spec, b_spec,
        ],
        out_specs=[row_spec,
                   pl.BlockSpec((BN, H), lambda i: (i, 0))],
        out_shape=[jax.ShapeDtypeStruct((N, H), jnp.float32),
                   jax.ShapeDtypeStruct((N, H), jnp.uint32)],
    )(x, msgs, w1a, w1b, b1n, w2n, b2n, wea, web, b1e)


def _edge_body(gsum_ref, e_ref, wc_ref, w2_ref, b2_ref, out_ref):
    eb = e_ref[...]
    p = gsum_ref[...]
    lo = lax.bitcast_convert_type(p << 16, jnp.float32)
    hi = lax.bitcast_convert_type(p & jnp.uint32(0xFFFF0000), jnp.float32)
    g = jnp.concatenate([lo, hi], axis=1)
    h = jnp.maximum(
        g + jnp.dot(eb, wc_ref[...], preferred_element_type=jnp.float32), 0.0)
    out_ref[...] = jnp.dot(h, w2_ref[...],
                           preferred_element_type=jnp.float32) \
        + b2_ref[...] + eb


def _edge_body_acc(gsum_ref, e_ref, wc_ref, w2_ref, b2_ref, prev_ref, out_ref):
    del prev_ref
    _edge_body(gsum_ref, e_ref, wc_ref, w2_ref, b2_ref, out_ref)


_BE = 2000


def _edge_tc(row0, gsum_k, e, wec, w2e, b2e, prev):
    blk0 = row0 // _BE
    nblk = gsum_k.shape[0] // _BE
    loc_spec = pl.BlockSpec((_BE, HP), lambda i: (i, 0))
    shift_spec = pl.BlockSpec((_BE, H), lambda i: (i + blk0, 0))
    w_spec = pl.BlockSpec((H, H), lambda i: (0, 0))
    b_spec = pl.BlockSpec((1, H), lambda i: (0, 0))
    body = _edge_body if prev is None else _edge_body_acc
    in_specs = [loc_spec, shift_spec, w_spec, w_spec, b_spec]
    args = [gsum_k, e, wec, w2e, b2e]
    aliases = {}
    if prev is not None:
        in_specs.append(pl.BlockSpec(memory_space=pl.ANY))
        args.append(prev)
        aliases = {5: 0}
    return pl.pallas_call(
        body,
        grid=(nblk,),
        in_specs=in_specs,
        out_specs=shift_spec,
        out_shape=jax.ShapeDtypeStruct((E, H), jnp.float32),
        input_output_aliases=aliases,
    )(*args)


def kernel(x, edge_index, e, W1n, b1n, W2n, b2n, W1e, b1e, W2e, b2e):
    end = edge_index[1].reshape(NC, NS, CPW, CH)
    zeros = jnp.zeros((N, H), jnp.float32)

    msgs = _seg_sum_sc(e, end, zeros)

    x_out, gpk_nodes = _node_tc(
        x, msgs,
        W1n[:H], W1n[H:], b1n.reshape(1, H), W2n, b2n.reshape(1, H),
        W1e[:H], W1e[H:2 * H], b1e.reshape(1, H))

    wec = W1e[2 * H:]
    b2e_r = b2e.reshape(1, H)
    gsums, row0s = [], []
    row0 = 0
    for cps in CPS_LIST:
        slc = cps * CH * NC * NS
        s_idx = lax.slice_in_dim(edge_index[0], row0, row0 + slc) \
            .reshape(NC, NS, cps, CH)
        e_idx = lax.slice_in_dim(edge_index[1], row0, row0 + slc) \
            .reshape(NC, NS, cps, CH)
        gsums.append(_make_gather_sc(cps)(gpk_nodes, s_idx, e_idx))
        row0s.append(row0)
        row0 += slc
    e_out = None
    for k in range(K):
        e_out = _edge_tc(row0s[k], gsums[k], e, wec, W2e, b2e_r, e_out)
    return (x_out, e_out)


# speedup vs baseline: 1.2051x; 1.0204x over previous
import functools

import jax
import jax.numpy as jnp
from jax import lax
from jax.experimental import pallas as pl
from jax.experimental.pallas import tpu as pltpu
from jax.experimental.pallas import tpu_sc as plsc

N, E, H = 10000, 320000, 128
NC, NS = 2, 16
CH = 80
EPW = E // (NC * NS)
CPW = EPW // CH
NPT = 624
NREM = N - NS * NPT
CPS_LIST = (25, 25, 25, 25, 25)
K = len(CPS_LIST)
HP = H // 2

_MESH = plsc.VectorSubcoreMesh(core_axis_name="c", subcore_axis_name="s")


@functools.partial(
    pl.kernel,
    mesh=_MESH,
    out_type=jax.ShapeDtypeStruct((NC, N, H), jnp.float32),
    scratch_types=[
        pltpu.VMEM((CPW, CH), jnp.int32),
        pltpu.VMEM((CH, H), jnp.float32),
        pltpu.VMEM((CH, H), jnp.float32),
        pltpu.VMEM_SHARED((N, H), jnp.float32),
        pltpu.SemaphoreType.DMA,
        pltpu.SemaphoreType.DMA,
    ],
)
def _seg_sum_sc(e_hbm, end_hbm, zeros_hbm, out_hbm, idx_v, rows0_v, rows1_v,
                acc_sh, sem0, sem1):
    c = lax.axis_index("c")
    s = lax.axis_index("s")
    pltpu.sync_copy(zeros_hbm.at[pl.ds(s * NPT, NPT)],
                    acc_sh.at[pl.ds(s * NPT, NPT)])

    @pl.when(s == NS - 1)
    def _zero_tail():
        pltpu.sync_copy(zeros_hbm.at[pl.ds(NS * NPT, NREM)],
                        acc_sh.at[pl.ds(NS * NPT, NREM)])

    pltpu.sync_copy(end_hbm.at[c, s], idx_v)
    plsc.subcore_barrier()
    wbase = (c * NS + s) * EPW

    def _start_in(i, rows, sem):
        pltpu.async_copy(e_hbm.at[pl.ds(wbase + i * CH, CH)], rows, sem)

    def _finish(i, rows, sem):
        pltpu.make_async_copy(e_hbm.at[pl.ds(wbase + i * CH, CH)],
                              rows, sem).wait()
        pltpu.sync_copy(rows, acc_sh.at[idx_v.at[i]], add=True)

    _start_in(0, rows0_v, sem0)

    def body(i, carry):
        b = lax.rem(i, 2)

        @pl.when(i + 1 < CPW)
        def _prefetch():
            @pl.when(b == 0)
            def _():
                _start_in(i + 1, rows1_v, sem1)

            @pl.when(b == 1)
            def _():
                _start_in(i + 1, rows0_v, sem0)

        @pl.when(b == 0)
        def _():
            _finish(i, rows0_v, sem0)

        @pl.when(b == 1)
        def _():
            _finish(i, rows1_v, sem1)

        return carry

    lax.fori_loop(0, CPW, body, 0)
    plsc.subcore_barrier()
    pltpu.sync_copy(acc_sh.at[pl.ds(s * NPT, NPT)],
                    out_hbm.at[c, pl.ds(s * NPT, NPT)])

    @pl.when(s == NS - 1)
    def _write_tail():
        pltpu.sync_copy(acc_sh.at[pl.ds(NS * NPT, NREM)],
                        out_hbm.at[c, pl.ds(NS * NPT, NREM)])


@functools.lru_cache(maxsize=None)
def _make_gather_sc(cps):
    cpt = cps * NC
    slc = cpt * CH * NS
    ept = cpt * CH

    @functools.partial(
        pl.kernel,
        mesh=_MESH,
        out_type=(jax.ShapeDtypeStruct((slc, HP), jnp.uint32),
                  jax.ShapeDtypeStruct((slc, HP), jnp.uint32)),
        scratch_types=[
            pltpu.VMEM((cpt, CH), jnp.int32),
            pltpu.VMEM((CH, HP), jnp.uint32),
            pltpu.VMEM((CH, HP), jnp.uint32),
            pltpu.VMEM_SHARED((N, HP), jnp.uint32),
            pltpu.SemaphoreType.DMA,
            pltpu.SemaphoreType.DMA,
        ],
    )
    def _gather_sc(ga_hbm, gb_hbm, start_hbm, end_hbm, outa_hbm, outb_hbm,
                   idx_v, buf0_v, buf1_v, tbl_sh, s0, s1):
        c = lax.axis_index("c")
        s = lax.axis_index("s")
        @pl.when(c == 0)
        def _stage_a():
            pltpu.sync_copy(ga_hbm.at[pl.ds(s * NPT, NPT)],
                            tbl_sh.at[pl.ds(s * NPT, NPT)])

            @pl.when(s == NS - 1)
            def _tail():
                pltpu.sync_copy(ga_hbm.at[pl.ds(NS * NPT, NREM)],
                                tbl_sh.at[pl.ds(NS * NPT, NREM)])

            pltpu.sync_copy(start_hbm.at[s], idx_v)

        @pl.when(c == 1)
        def _stage_b():
            pltpu.sync_copy(gb_hbm.at[pl.ds(s * NPT, NPT)],
                            tbl_sh.at[pl.ds(s * NPT, NPT)])

            @pl.when(s == NS - 1)
            def _tail():
                pltpu.sync_copy(gb_hbm.at[pl.ds(NS * NPT, NREM)],
                                tbl_sh.at[pl.ds(NS * NPT, NREM)])

            pltpu.sync_copy(end_hbm.at[s], idx_v)

        plsc.subcore_barrier()
        wbase = s * ept

        def _start_in(i, buf, sem):
            pltpu.async_copy(tbl_sh.at[idx_v.at[i]], buf, sem)

        def _finish(i, buf, sem):
            pltpu.make_async_copy(tbl_sh.at[idx_v.at[i]], buf, sem).wait()
            base = wbase + i * CH

            @pl.when(c == 0)
            def _():
                pltpu.sync_copy(buf, outa_hbm.at[pl.ds(base, CH)])

            @pl.when(c == 1)
            def _():
                pltpu.sync_copy(buf, outb_hbm.at[pl.ds(base, CH)])

        _start_in(0, buf0_v, s0)

        def body(i, carry):
            b = lax.rem(i, 2)

            @pl.when(i + 1 < cpt)
            def _prefetch():
                @pl.when(b == 0)
                def _():
                    _start_in(i + 1, buf1_v, s1)

                @pl.when(b == 1)
                def _():
                    _start_in(i + 1, buf0_v, s0)

            @pl.when(b == 0)
            def _():
                _finish(i, buf0_v, s0)

            @pl.when(b == 1)
            def _():
                _finish(i, buf1_v, s1)

            return carry

        lax.fori_loop(0, cpt, body, 0)

    return _gather_sc


def _pack_bf16(v):
    u = lax.bitcast_convert_type(v, jnp.uint32)
    r = (u + jnp.uint32(0x7FFF) + ((u >> 16) & jnp.uint32(1))) >> 16
    return r[:, :HP] | (r[:, HP:] << 16)


def _node_body(x_ref, m_ref, w1a_ref, w1b_ref, b1_ref, w2_ref, b2_ref,
               wa_ref, wb_ref, be_ref, xout_ref, ga_ref, gb_ref):
    xb = x_ref[...]
    m = m_ref[0] + m_ref[1]
    h = jnp.maximum(
        jnp.dot(xb, w1a_ref[...], preferred_element_type=jnp.float32)
        + jnp.dot(m, w1b_ref[...], preferred_element_type=jnp.float32)
        + b1_ref[...], 0.0)
    xo = jnp.dot(h, w2_ref[...], preferred_element_type=jnp.float32) \
        + b2_ref[...] + xb
    xout_ref[...] = xo
    ga_ref[...] = _pack_bf16(
        jnp.dot(xo, wa_ref[...], preferred_element_type=jnp.float32)
        + be_ref[...])
    gb_ref[...] = _pack_bf16(
        jnp.dot(xo, wb_ref[...], preferred_element_type=jnp.float32))


def _node_tc(x, msgs, w1a, w1b, b1n, w2n, b2n, wea, web, b1e):
    BN = 1000
    grid = (N // BN,)
    row_spec = pl.BlockSpec((BN, H), lambda i: (i, 0))
    w_spec = pl.BlockSpec((H, H), lambda i: (0, 0))
    b_spec = pl.BlockSpec((1, H), lambda i: (0, 0))
    return pl.pallas_call(
        _node_body,
        grid=grid,
        in_specs=[
            row_spec,
            pl.BlockSpec((NC, BN, H), lambda i: (0, i, 0)),
            w_spec, w_spec, b_spec, w_spec, b_spec, w_spec, w_spec, b_spec,
        ],
        out_specs=[row_spec,
                   pl.BlockSpec((BN, HP), lambda i: (i, 0)),
                   pl.BlockSpec((BN, HP), lambda i: (i, 0))],
        out_shape=[jax.ShapeDtypeStruct((N, H), jnp.float32),
                   jax.ShapeDtypeStruct((N, HP), jnp.uint32),
                   jax.ShapeDtypeStruct((N, HP), jnp.uint32)],
    )(x, msgs, w1a, w1b, b1n, w2n, b2n, wea, web, b1e)


def _unpack_bf16(p):
    lo = lax.bitcast_convert_type(p << 16, jnp.float32)
    hi = lax.bitcast_convert_type(p & jnp.uint32(0xFFFF0000), jnp.float32)
    return jnp.concatenate([lo, hi], axis=1)


def _edge_body(ga_ref, gb_ref, e_ref, wc_ref, w2_ref, b2_ref, out_ref):
    eb = e_ref[...]
    g = _unpack_bf16(ga_ref[...]) + _unpack_bf16(gb_ref[...])
    h = jnp.maximum(
        g + jnp.dot(eb, wc_ref[...], preferred_element_type=jnp.float32), 0.0)
    out_ref[...] = jnp.dot(h, w2_ref[...],
                           preferred_element_type=jnp.float32) \
        + b2_ref[...] + eb


def _edge_body_acc(ga_ref, gb_ref, e_ref, wc_ref, w2_ref, b2_ref,
                   prev_ref, out_ref):
    del prev_ref
    _edge_body(ga_ref, gb_ref, e_ref, wc_ref, w2_ref, b2_ref, out_ref)


_BE = 2000


def _edge_tc(row0, ga_k, gb_k, e, wec, w2e, b2e, prev):
    blk0 = row0 // _BE
    nblk = ga_k.shape[0] // _BE
    loc_spec = pl.BlockSpec((_BE, HP), lambda i: (i, 0))
    shift_spec = pl.BlockSpec((_BE, H), lambda i: (i + blk0, 0))
    w_spec = pl.BlockSpec((H, H), lambda i: (0, 0))
    b_spec = pl.BlockSpec((1, H), lambda i: (0, 0))
    body = _edge_body if prev is None else _edge_body_acc
    in_specs = [loc_spec, loc_spec, shift_spec, w_spec, w_spec, b_spec]
    args = [ga_k, gb_k, e, wec, w2e, b2e]
    aliases = {}
    if prev is not None:
        in_specs.append(pl.BlockSpec(memory_space=pl.ANY))
        args.append(prev)
        aliases = {6: 0}
    return pl.pallas_call(
        body,
        grid=(nblk,),
        in_specs=in_specs,
        out_specs=shift_spec,
        out_shape=jax.ShapeDtypeStruct((E, H), jnp.float32),
        input_output_aliases=aliases,
    )(*args)


def kernel(x, edge_index, e, W1n, b1n, W2n, b2n, W1e, b1e, W2e, b2e):
    end = edge_index[1].reshape(NC, NS, CPW, CH)
    zeros = jnp.zeros((N, H), jnp.float32)

    msgs = _seg_sum_sc(e, end, zeros)

    x_out, ga_nodes, gb_nodes = _node_tc(
        x, msgs,
        W1n[:H], W1n[H:], b1n.reshape(1, H), W2n, b2n.reshape(1, H),
        W1e[:H], W1e[H:2 * H], b1e.reshape(1, H))

    wec = W1e[2 * H:]
    b2e_r = b2e.reshape(1, H)
    gpairs, row0s = [], []
    row0 = 0
    for cps in CPS_LIST:
        cpt = cps * NC
        slc = cpt * CH * NS
        s_idx = lax.slice_in_dim(edge_index[0], row0, row0 + slc) \
            .reshape(NS, cpt, CH)
        e_idx = lax.slice_in_dim(edge_index[1], row0, row0 + slc) \
            .reshape(NS, cpt, CH)
        gpairs.append(_make_gather_sc(cps)(ga_nodes, gb_nodes, s_idx, e_idx))
        row0s.append(row0)
        row0 += slc
    e_out = None
    for k in range(K):
        ga_k, gb_k = gpairs[k]
        e_out = _edge_tc(row0s[k], ga_k, gb_k, e, wec, W2e, b2e_r, e_out)
    return (x_out, e_out)
